# Initial kernel scaffold; baseline (speedup 1.0000x reference)
import functools

import jax
import jax.numpy as jnp
from jax import lax
from jax.experimental import pallas as pl
from jax.experimental.pallas import tpu as pltpu, tpu_sc as plsc

N = 10000
E = 320000
D_IN = 128
D_HID = 128
D_OUT = 64
BN_EPS = 1e-3
ALPHA = 0.2

NC = 2
NS = 16
NW = NC * NS
K = 80
NCH = E // (NW * K)
NPT = N // NS
ZR = 125
DEGW = 16

_Z16 = jnp.zeros((16,), jnp.float32)


def _make_sc_agg(d, with_deg):
    mesh = plsc.VectorSubcoreMesh(core_axis_name="c", subcore_axis_name="s")
    out_type = [jax.ShapeDtypeStruct((NC, N, d), jnp.float32)]
    scratch = [
        pltpu.VMEM((NCH, K), jnp.int32),
        pltpu.VMEM((NCH, K), jnp.int32),
        pltpu.VMEM((K, d), jnp.float32),
        pltpu.VMEM((ZR, d), jnp.float32),
        pltpu.VMEM_SHARED((N, d), jnp.float32),
        pltpu.SemaphoreType.DMA,
    ]
    if with_deg:
        out_type.append(jax.ShapeDtypeStruct((NC, N, DEGW), jnp.float32))
        scratch += [
            pltpu.VMEM((K, DEGW), jnp.float32),
            pltpu.VMEM((NPT, DEGW), jnp.float32),
            pltpu.VMEM_SHARED((N, DEGW), jnp.float32),
        ]

    def body(table, src_h, dst_h, *rest):
        if with_deg:
            (out_agg, out_deg, src_v, dst_v, rows_v, zbuf, agg_sh, sem,
             ones_v, dzbuf, deg_sh) = rest
        else:
            out_agg, src_v, dst_v, rows_v, zbuf, agg_sh, sem = rest
        cid = lax.axis_index("c")
        sid = lax.axis_index("s")
        wid = sid * NC + cid

        pltpu.sync_copy(src_h.at[pl.ds(wid * NCH, NCH)], src_v)
        pltpu.sync_copy(dst_h.at[pl.ds(wid * NCH, NCH)], dst_v)

        def zrow(r, _):
            for c in range(d // 16):
                zbuf[r, pl.ds(c * 16, 16)] = _Z16
            return 0
        lax.fori_loop(0, ZR, zrow, 0)
        base = sid * NPT
        for t in range(NPT // ZR):
            pltpu.sync_copy(zbuf, agg_sh.at[pl.ds(base + t * ZR, ZR)])
        if with_deg:
            one16 = jnp.ones((16,), jnp.float32)

            def orow(r, _):
                ones_v[r] = one16
                return 0
            lax.fori_loop(0, K, orow, 0)

            def dzrow(r, _):
                dzbuf[r] = _Z16
                return 0
            lax.fori_loop(0, NPT, dzrow, 0)
            pltpu.sync_copy(dzbuf, deg_sh.at[pl.ds(base, NPT)])
        plsc.subcore_barrier()

        def chunk(j, _):
            pltpu.async_copy(table.at[src_v.at[j]], rows_v, sem).wait()
            pltpu.sync_copy(rows_v, agg_sh.at[dst_v.at[j]], add=True)
            if with_deg:
                pltpu.sync_copy(ones_v, deg_sh.at[dst_v.at[j]], add=True)
            return 0
        lax.fori_loop(0, NCH, chunk, 0)
        plsc.subcore_barrier()

        for t in range(NPT // ZR):
            off = base + t * ZR
            pltpu.sync_copy(agg_sh.at[pl.ds(off, ZR)],
                            out_agg.at[cid, pl.ds(off, ZR)])
        if with_deg:
            pltpu.sync_copy(deg_sh.at[pl.ds(base, NPT)],
                            out_deg.at[cid, pl.ds(base, NPT)])

    return functools.partial(
        pl.kernel, body, out_type=out_type, mesh=mesh, scratch_types=scratch)


_sc_agg_l1 = _make_sc_agg(D_IN, True)
_sc_agg_l2 = _make_sc_agg(D_OUT, False)

_HI = lax.Precision.HIGHEST


def _dense1(x, aggp, degp, A1, B1, c1, B2):
    R = 1000
    G = N // R

    def body(x_ref, agg_ref, deg_ref, a_ref, b_ref, c_ref, w2_ref,
             h_ref, hw_ref):
        agg = agg_ref[0] + agg_ref[1]
        deg = deg_ref[0][:, 0:1] + deg_ref[1][:, 0:1]
        mean = agg / jnp.maximum(deg, 1.0)
        h = (jnp.dot(x_ref[...], a_ref[...], precision=_HI,
                     preferred_element_type=jnp.float32)
             + jnp.dot(mean, b_ref[...], precision=_HI,
                       preferred_element_type=jnp.float32)
             + c_ref[...])
        h = jnp.where(h >= 0, h, ALPHA * h)
        h_ref[...] = h
        hw_ref[...] = jnp.dot(h, w2_ref[...], precision=_HI,
                              preferred_element_type=jnp.float32)

    return pl.pallas_call(
        body,
        grid=(G,),
        in_specs=[
            pl.BlockSpec((R, D_IN), lambda i: (i, 0)),
            pl.BlockSpec((NC, R, D_HID), lambda i: (0, i, 0)),
            pl.BlockSpec((NC, R, DEGW), lambda i: (0, i, 0)),
            pl.BlockSpec((D_IN, D_HID), lambda i: (0, 0)),
            pl.BlockSpec((D_IN, D_HID), lambda i: (0, 0)),
            pl.BlockSpec((1, D_HID), lambda i: (0, 0)),
            pl.BlockSpec((D_HID, D_OUT), lambda i: (0, 0)),
        ],
        out_specs=[
            pl.BlockSpec((R, D_HID), lambda i: (i, 0)),
            pl.BlockSpec((R, D_OUT), lambda i: (i, 0)),
        ],
        out_shape=[
            jax.ShapeDtypeStruct((N, D_HID), jnp.float32),
            jax.ShapeDtypeStruct((N, D_OUT), jnp.float32),
        ],
    )(x, aggp, degp, A1, B1, c1, B2)


def _dense2(h, aggp2, degp, A2, c2):
    R = 1000
    G = N // R

    def body(h_ref, agg_ref, deg_ref, a_ref, c_ref, o_ref):
        agg = agg_ref[0] + agg_ref[1]
        deg = deg_ref[0][:, 0:1] + deg_ref[1][:, 0:1]
        mean = agg / jnp.maximum(deg, 1.0)
        y = (jnp.dot(h_ref[...], a_ref[...], precision=_HI,
                     preferred_element_type=jnp.float32)
             + mean + c_ref[...])
        o_ref[...] = jnp.where(y >= 0, y, ALPHA * y)

    return pl.pallas_call(
        body,
        grid=(G,),
        in_specs=[
            pl.BlockSpec((R, D_HID), lambda i: (i, 0)),
            pl.BlockSpec((NC, R, D_OUT), lambda i: (0, i, 0)),
            pl.BlockSpec((NC, R, DEGW), lambda i: (0, i, 0)),
            pl.BlockSpec((D_HID, D_OUT), lambda i: (0, 0)),
            pl.BlockSpec((1, D_OUT), lambda i: (0, 0)),
        ],
        out_specs=pl.BlockSpec((R, D_OUT), lambda i: (i, 0)),
        out_shape=jax.ShapeDtypeStruct((N, D_OUT), jnp.float32),
    )(h, aggp2, degp, A2, c2)


def kernel(x, edge_index, W_self1, W_neigh1, b1, gamma1, beta1,
           W_self2, W_neigh2, b2, gamma2, beta2):
    s1 = gamma1 * lax.rsqrt(jnp.float32(1.0 + BN_EPS))
    A1 = W_self1 * s1[None, :]
    B1 = W_neigh1 * s1[None, :]
    c1 = (b1 * s1 + beta1)[None, :]
    s2 = gamma2 * lax.rsqrt(jnp.float32(1.0 + BN_EPS))
    A2 = W_self2 * s2[None, :]
    B2 = W_neigh2 * s2[None, :]
    c2 = (b2 * s2 + beta2)[None, :]

    src = edge_index[0].astype(jnp.int32).reshape(E // K, K)
    dst = edge_index[1].astype(jnp.int32).reshape(E // K, K)

    aggp1, degp = _sc_agg_l1(x, src, dst)
    h, hw = _dense1(x, aggp1, degp, A1, B1, c1, B2)
    aggp2 = _sc_agg_l2(hw, src, dst)
    out = _dense2(h, aggp2, degp, A2, c2)
    return out


# speedup vs baseline: 4.1588x; 4.1588x over previous
import jax
import jax.numpy as jnp
from jax import lax
from jax.experimental import pallas as pl
from jax.experimental.pallas import tpu as pltpu, tpu_sc as plsc

N = 10000
E = 320000
D_IN = 128
D_HID = 128
D_OUT = 64
BN_EPS = 1e-3
ALPHA = 0.2

NC = 2
NS = 16
NW = NC * NS
K = 80
CH = 104
NFULL = N // CH
TAIL = N - NFULL * CH
DEGW = 16


def _make_sc_agg(d, split, with_deg):
    ept = E // NS if split else E // NW
    ncht = ept // K
    mesh = plsc.VectorSubcoreMesh(core_axis_name="c", subcore_axis_name="s")
    out_type = [jax.ShapeDtypeStruct((NC, N, d), jnp.float32)]
    scratch = [
        pltpu.VMEM((K,), jnp.int32),
        pltpu.VMEM((K,), jnp.int32),
        pltpu.VMEM((K, d), jnp.float32),
        pltpu.VMEM((CH, d), jnp.float32),
        pltpu.VMEM_SHARED((N, d), jnp.float32),
        pltpu.SemaphoreType.DMA,
    ]
    if with_deg:
        out_type.append(jax.ShapeDtypeStruct((NC, N, DEGW), jnp.float32))
        scratch += [
            pltpu.VMEM((K, DEGW), jnp.float32),
            pltpu.VMEM((CH, DEGW), jnp.float32),
            pltpu.VMEM_SHARED((N, DEGW), jnp.float32),
        ]

    def body(table, src_h, dst_h, *rest):
        if with_deg:
            (out_agg, out_deg, src_v, dst_v, rows_v, zbuf, agg_sh, sem,
             ones_v, dzbuf, deg_sh) = rest
        else:
            out_agg, src_v, dst_v, rows_v, zbuf, agg_sh, sem = rest
        cid = lax.axis_index("c")
        sid = lax.axis_index("s")
        z16 = jnp.zeros((16,), jnp.float32)
        tbl = table.at[cid] if split else table

        def zrow(r, _):
            for c in range(d // 16):
                zbuf[r, pl.ds(c * 16, 16)] = z16
            return 0
        lax.fori_loop(0, CH, zrow, 0)
        for t in range(NFULL // NS):
            r0 = (sid + NS * t) * CH
            pltpu.sync_copy(zbuf, agg_sh.at[pl.ds(r0, CH)])

        @pl.when(sid == 0)
        def _zero_tail():
            pltpu.sync_copy(zbuf.at[pl.ds(0, TAIL)],
                            agg_sh.at[pl.ds(NFULL * CH, TAIL)])
        if with_deg:
            one16 = jnp.ones((16,), jnp.float32)

            def orow(r, _):
                ones_v[r] = one16
                return 0
            lax.fori_loop(0, K, orow, 0)

            def dzrow(r, _):
                dzbuf[r] = z16
                return 0
            lax.fori_loop(0, CH, dzrow, 0)
            for t in range(NFULL // NS):
                r0 = (sid + NS * t) * CH
                pltpu.sync_copy(dzbuf, deg_sh.at[pl.ds(r0, CH)])

            @pl.when(sid == 0)
            def _zero_deg_tail():
                pltpu.sync_copy(dzbuf.at[pl.ds(0, TAIL)],
                                deg_sh.at[pl.ds(NFULL * CH, TAIL)])
        plsc.subcore_barrier()

        if split:
            ebase = sid * ept
        else:
            ebase = (sid * NC + cid) * ept

        def chunk(j, _):
            off = ebase + j * K
            pltpu.sync_copy(src_h.at[pl.ds(off, K)], src_v)
            pltpu.sync_copy(dst_h.at[pl.ds(off, K)], dst_v)
            pltpu.async_copy(tbl.at[src_v], rows_v, sem).wait()
            pltpu.sync_copy(rows_v, agg_sh.at[dst_v], add=True)
            if with_deg:
                pltpu.sync_copy(ones_v, deg_sh.at[dst_v], add=True)
            return 0
        lax.fori_loop(0, ncht, chunk, 0)
        plsc.subcore_barrier()

        for t in range(NFULL // NS):
            r0 = (sid + NS * t) * CH
            pltpu.sync_copy(agg_sh.at[pl.ds(r0, CH)],
                            out_agg.at[cid, pl.ds(r0, CH)])

        @pl.when(sid == 0)
        def _write_tail():
            pltpu.sync_copy(agg_sh.at[pl.ds(NFULL * CH, TAIL)],
                            out_agg.at[cid, pl.ds(NFULL * CH, TAIL)])
        if with_deg:
            for t in range(NFULL // NS):
                r0 = (sid + NS * t) * CH
                pltpu.sync_copy(deg_sh.at[pl.ds(r0, CH)],
                                out_deg.at[cid, pl.ds(r0, CH)])

            @pl.when(sid == 0)
            def _write_deg_tail():
                pltpu.sync_copy(deg_sh.at[pl.ds(NFULL * CH, TAIL)],
                                out_deg.at[cid, pl.ds(NFULL * CH, TAIL)])

    return pl.kernel(
        body, out_type=out_type, mesh=mesh, scratch_types=scratch,
        compiler_params=pltpu.CompilerParams(use_tc_tiling_on_sc=False))


_sc_agg_l1 = _make_sc_agg(D_IN // 2, True, True)
_sc_agg_l2 = _make_sc_agg(D_OUT, False, False)

_HI = lax.Precision.HIGHEST


def _dot(a, b):
    return jnp.dot(a, b, precision=_HI, preferred_element_type=jnp.float32)


def _dense1(x, aggp, degp, A1, B1a, B1b, c1, B2):
    R = 1000
    G = N // R

    def body(x_ref, agg_ref, deg_ref, a_ref, b1a_ref, b1b_ref, c_ref,
             w2_ref, h_ref, hw_ref):
        deg = jnp.maximum(deg_ref[0][:, 0:1], 1.0)
        m0 = agg_ref[0] / deg
        m1 = agg_ref[1] / deg
        h = (_dot(x_ref[...], a_ref[...]) + _dot(m0, b1a_ref[...])
             + _dot(m1, b1b_ref[...]) + c_ref[...])
        h = jnp.where(h >= 0, h, ALPHA * h)
        h_ref[...] = h
        hw_ref[...] = _dot(h, w2_ref[...])

    return pl.pallas_call(
        body,
        grid=(G,),
        in_specs=[
            pl.BlockSpec((R, D_IN), lambda i: (i, 0)),
            pl.BlockSpec((NC, R, D_IN // 2), lambda i: (0, i, 0)),
            pl.BlockSpec((1, R, DEGW), lambda i: (0, i, 0)),
            pl.BlockSpec((D_IN, D_HID), lambda i: (0, 0)),
            pl.BlockSpec((D_IN // 2, D_HID), lambda i: (0, 0)),
            pl.BlockSpec((D_IN // 2, D_HID), lambda i: (0, 0)),
            pl.BlockSpec((1, D_HID), lambda i: (0, 0)),
            pl.BlockSpec((D_HID, D_OUT), lambda i: (0, 0)),
        ],
        out_specs=[
            pl.BlockSpec((R, D_HID), lambda i: (i, 0)),
            pl.BlockSpec((R, D_OUT), lambda i: (i, 0)),
        ],
        out_shape=[
            jax.ShapeDtypeStruct((N, D_HID), jnp.float32),
            jax.ShapeDtypeStruct((N, D_OUT), jnp.float32),
        ],
    )(x, aggp, degp, A1, B1a, B1b, c1, B2)


def _dense2(h, aggp2, degp, A2, c2):
    R = 1000
    G = N // R

    def body(h_ref, agg_ref, deg_ref, a_ref, c_ref, o_ref):
        deg = jnp.maximum(deg_ref[0][:, 0:1], 1.0)
        mean = (agg_ref[0] + agg_ref[1]) / deg
        y = _dot(h_ref[...], a_ref[...]) + mean + c_ref[...]
        o_ref[...] = jnp.where(y >= 0, y, ALPHA * y)

    return pl.pallas_call(
        body,
        grid=(G,),
        in_specs=[
            pl.BlockSpec((R, D_HID), lambda i: (i, 0)),
            pl.BlockSpec((NC, R, D_OUT), lambda i: (0, i, 0)),
            pl.BlockSpec((1, R, DEGW), lambda i: (0, i, 0)),
            pl.BlockSpec((D_HID, D_OUT), lambda i: (0, 0)),
            pl.BlockSpec((1, D_OUT), lambda i: (0, 0)),
        ],
        out_specs=pl.BlockSpec((R, D_OUT), lambda i: (i, 0)),
        out_shape=jax.ShapeDtypeStruct((N, D_OUT), jnp.float32),
    )(h, aggp2, degp, A2, c2)


def kernel(x, edge_index, W_self1, W_neigh1, b1, gamma1, beta1,
           W_self2, W_neigh2, b2, gamma2, beta2):
    s1 = gamma1 * lax.rsqrt(jnp.float32(1.0 + BN_EPS))
    A1 = W_self1 * s1[None, :]
    B1 = W_neigh1 * s1[None, :]
    c1 = (b1 * s1 + beta1)[None, :]
    s2 = gamma2 * lax.rsqrt(jnp.float32(1.0 + BN_EPS))
    A2 = W_self2 * s2[None, :]
    B2 = W_neigh2 * s2[None, :]
    c2 = (b2 * s2 + beta2)[None, :]

    src = edge_index[0].astype(jnp.int32)
    dst = edge_index[1].astype(jnp.int32)

    xs = x.reshape(N, NC, D_IN // 2).transpose(1, 0, 2)

    aggp1, degp = _sc_agg_l1(xs, src, dst)
    h, hw = _dense1(x, aggp1, degp, A1, B1[: D_IN // 2], B1[D_IN // 2:],
                    c1, B2)
    [aggp2] = _sc_agg_l2(hw, src, dst)
    out = _dense2(h, aggp2, degp, A2, c2)
    return out


# speedup vs baseline: 11.4785x; 2.7601x over previous
import jax
import jax.numpy as jnp
from jax import lax
from jax.experimental import pallas as pl
from jax.experimental.pallas import tpu as pltpu, tpu_sc as plsc

N = 10000
E = 320000
D_IN = 128
D_HID = 128
D_OUT = 64
BN_EPS = 1e-3
ALPHA = 0.2

NC = 2
NS = 16
NW = NC * NS
NB = 5
K = 80
CH = 104
NFULL = N // CH
TAIL = N - NFULL * CH
DEGW = 16


def _make_sc_agg(d, split, with_deg):
    ept = E // NS if split else E // NW
    ncht = ept // K
    assert ncht % NB == 0
    mesh = plsc.VectorSubcoreMesh(core_axis_name="c", subcore_axis_name="s")
    out_type = [jax.ShapeDtypeStruct((NC, N, d), jnp.float32)]
    scratch = (
        [pltpu.VMEM((ncht, K), jnp.int32),
         pltpu.VMEM((ncht, K), jnp.int32),
         pltpu.VMEM((CH, d), jnp.float32),
         pltpu.VMEM_SHARED((N, d), jnp.float32)]
        + [pltpu.VMEM((K, d), jnp.float32)] * NB
        + [pltpu.SemaphoreType.DMA] * (2 * NB)
    )
    if with_deg:
        out_type.append(jax.ShapeDtypeStruct((NC, N, DEGW), jnp.float32))
        scratch += (
            [pltpu.VMEM((K, DEGW), jnp.float32),
             pltpu.VMEM((CH, DEGW), jnp.float32),
             pltpu.VMEM_SHARED((N, DEGW), jnp.float32)]
            + [pltpu.SemaphoreType.DMA] * NB
        )

    def body(table, src_h, dst_h, *rest):
        n_out = 2 if with_deg else 1
        if with_deg:
            out_agg, out_deg = rest[:2]
        else:
            (out_agg,) = rest[:1]
        rest = rest[n_out:]
        src_a, dst_a, zbuf, agg_sh = rest[:4]
        bufs = rest[4:4 + NB]
        gsem = rest[4 + NB:4 + 2 * NB]
        ssem = rest[4 + 2 * NB:4 + 3 * NB]
        if with_deg:
            ones_v, dzbuf, deg_sh = rest[4 + 3 * NB:7 + 3 * NB]
            dsem = rest[7 + 3 * NB:7 + 4 * NB]
        cid = lax.axis_index("c")
        sid = lax.axis_index("s")
        z16 = jnp.zeros((16,), jnp.float32)
        tbl = table.at[cid] if split else table

        if split:
            crow = sid * ncht
        else:
            crow = (sid * NC + cid) * ncht
        pltpu.sync_copy(src_h.at[pl.ds(crow, ncht)], src_a)
        pltpu.sync_copy(dst_h.at[pl.ds(crow, ncht)], dst_a)

        def zrow(r, _):
            for c in range(d // 16):
                zbuf[r, pl.ds(c * 16, 16)] = z16
            return 0
        lax.fori_loop(0, CH, zrow, 0)
        for t in range(NFULL // NS):
            r0 = (sid + NS * t) * CH
            pltpu.sync_copy(zbuf, agg_sh.at[pl.ds(r0, CH)])

        @pl.when(sid == 0)
        def _zero_tail():
            pltpu.sync_copy(zbuf.at[pl.ds(0, TAIL)],
                            agg_sh.at[pl.ds(NFULL * CH, TAIL)])
        if with_deg:
            one16 = jnp.ones((16,), jnp.float32)

            def orow(r, _):
                ones_v[r] = one16
                return 0
            lax.fori_loop(0, K, orow, 0)

            def dzrow(r, _):
                dzbuf[r] = z16
                return 0
            lax.fori_loop(0, CH, dzrow, 0)
            for t in range(NFULL // NS):
                r0 = (sid + NS * t) * CH
                pltpu.sync_copy(dzbuf, deg_sh.at[pl.ds(r0, CH)])

            @pl.when(sid == 0)
            def _zero_deg_tail():
                pltpu.sync_copy(dzbuf.at[pl.ds(0, TAIL)],
                                deg_sh.at[pl.ds(NFULL * CH, TAIL)])
        plsc.subcore_barrier()

        for b in range(NB):
            pltpu.async_copy(tbl.at[src_a.at[b]], bufs[b], gsem[b])

        def blk(i, _):
            base = i * NB
            for b in range(NB):
                j = base + b
                pltpu.make_async_copy(tbl.at[src_a.at[j]], bufs[b],
                                      gsem[b]).wait()
                pltpu.async_copy(bufs[b], agg_sh.at[dst_a.at[j]], ssem[b],
                                 add=True)
                if with_deg:
                    pltpu.async_copy(ones_v, deg_sh.at[dst_a.at[j]],
                                     dsem[b], add=True)
            for b in range(NB):
                j = base + b
                pltpu.make_async_copy(bufs[b], agg_sh.at[dst_a.at[j]],
                                      ssem[b]).wait()
                if with_deg:
                    pltpu.make_async_copy(ones_v, deg_sh.at[dst_a.at[j]],
                                          dsem[b]).wait()
                jn = j + NB

                @pl.when(jn < ncht)
                def _next_gather():
                    pltpu.async_copy(tbl.at[src_a.at[jn]], bufs[b], gsem[b])
            return 0
        lax.fori_loop(0, ncht // NB, blk, 0)
        plsc.subcore_barrier()

        for t in range(NFULL // NS):
            r0 = (sid + NS * t) * CH
            pltpu.sync_copy(agg_sh.at[pl.ds(r0, CH)],
                            out_agg.at[cid, pl.ds(r0, CH)])

        @pl.when(sid == 0)
        def _write_tail():
            pltpu.sync_copy(agg_sh.at[pl.ds(NFULL * CH, TAIL)],
                            out_agg.at[cid, pl.ds(NFULL * CH, TAIL)])
        if with_deg:
            for t in range(NFULL // NS):
                r0 = (sid + NS * t) * CH
                pltpu.sync_copy(deg_sh.at[pl.ds(r0, CH)],
                                out_deg.at[cid, pl.ds(r0, CH)])

            @pl.when(sid == 0)
            def _write_deg_tail():
                pltpu.sync_copy(deg_sh.at[pl.ds(NFULL * CH, TAIL)],
                                out_deg.at[cid, pl.ds(NFULL * CH, TAIL)])

    return pl.kernel(
        body, out_type=out_type, mesh=mesh, scratch_types=scratch,
        compiler_params=pltpu.CompilerParams(use_tc_tiling_on_sc=False))


_sc_agg_l1 = _make_sc_agg(D_IN // 2, True, True)
_sc_agg_l2 = _make_sc_agg(D_OUT, False, False)

_HI = lax.Precision.HIGHEST


def _dot(a, b):
    return jnp.dot(a, b, precision=_HI, preferred_element_type=jnp.float32)


def _dense1(x, aggp, degp, A1, B1a, B1b, c1, B2):
    R = 1000
    G = N // R

    def body(x_ref, agg_ref, deg_ref, a_ref, b1a_ref, b1b_ref, c_ref,
             w2_ref, h_ref, hw_ref):
        deg = jnp.maximum(deg_ref[0][:, 0:1], 1.0)
        m0 = agg_ref[0] / deg
        m1 = agg_ref[1] / deg
        h = (_dot(x_ref[...], a_ref[...]) + _dot(m0, b1a_ref[...])
             + _dot(m1, b1b_ref[...]) + c_ref[...])
        h = jnp.where(h >= 0, h, ALPHA * h)
        h_ref[...] = h
        hw_ref[...] = _dot(h, w2_ref[...])

    return pl.pallas_call(
        body,
        grid=(G,),
        in_specs=[
            pl.BlockSpec((R, D_IN), lambda i: (i, 0)),
            pl.BlockSpec((NC, R, D_IN // 2), lambda i: (0, i, 0)),
            pl.BlockSpec((1, R, DEGW), lambda i: (0, i, 0)),
            pl.BlockSpec((D_IN, D_HID), lambda i: (0, 0)),
            pl.BlockSpec((D_IN // 2, D_HID), lambda i: (0, 0)),
            pl.BlockSpec((D_IN // 2, D_HID), lambda i: (0, 0)),
            pl.BlockSpec((1, D_HID), lambda i: (0, 0)),
            pl.BlockSpec((D_HID, D_OUT), lambda i: (0, 0)),
        ],
        out_specs=[
            pl.BlockSpec((R, D_HID), lambda i: (i, 0)),
            pl.BlockSpec((R, D_OUT), lambda i: (i, 0)),
        ],
        out_shape=[
            jax.ShapeDtypeStruct((N, D_HID), jnp.float32),
            jax.ShapeDtypeStruct((N, D_OUT), jnp.float32),
        ],
    )(x, aggp, degp, A1, B1a, B1b, c1, B2)


def _dense2(h, aggp2, degp, A2, c2):
    R = 1000
    G = N // R

    def body(h_ref, agg_ref, deg_ref, a_ref, c_ref, o_ref):
        deg = jnp.maximum(deg_ref[0][:, 0:1], 1.0)
        mean = (agg_ref[0] + agg_ref[1]) / deg
        y = _dot(h_ref[...], a_ref[...]) + mean + c_ref[...]
        o_ref[...] = jnp.where(y >= 0, y, ALPHA * y)

    return pl.pallas_call(
        body,
        grid=(G,),
        in_specs=[
            pl.BlockSpec((R, D_HID), lambda i: (i, 0)),
            pl.BlockSpec((NC, R, D_OUT), lambda i: (0, i, 0)),
            pl.BlockSpec((1, R, DEGW), lambda i: (0, i, 0)),
            pl.BlockSpec((D_HID, D_OUT), lambda i: (0, 0)),
            pl.BlockSpec((1, D_OUT), lambda i: (0, 0)),
        ],
        out_specs=pl.BlockSpec((R, D_OUT), lambda i: (i, 0)),
        out_shape=jax.ShapeDtypeStruct((N, D_OUT), jnp.float32),
    )(h, aggp2, degp, A2, c2)


def kernel(x, edge_index, W_self1, W_neigh1, b1, gamma1, beta1,
           W_self2, W_neigh2, b2, gamma2, beta2):
    s1 = gamma1 * lax.rsqrt(jnp.float32(1.0 + BN_EPS))
    A1 = W_self1 * s1[None, :]
    B1 = W_neigh1 * s1[None, :]
    c1 = (b1 * s1 + beta1)[None, :]
    s2 = gamma2 * lax.rsqrt(jnp.float32(1.0 + BN_EPS))
    A2 = W_self2 * s2[None, :]
    B2 = W_neigh2 * s2[None, :]
    c2 = (b2 * s2 + beta2)[None, :]

    src = edge_index[0].astype(jnp.int32).reshape(E // K, K)
    dst = edge_index[1].astype(jnp.int32).reshape(E // K, K)

    xs = x.reshape(N, NC, D_IN // 2).transpose(1, 0, 2)

    aggp1, degp = _sc_agg_l1(xs, src, dst)
    h, hw = _dense1(x, aggp1, degp, A1, B1[: D_IN // 2], B1[D_IN // 2:],
                    c1, B2)
    [aggp2] = _sc_agg_l2(hw, src, dst)
    out = _dense2(h, aggp2, degp, A2, c2)
    return out


# speedup vs baseline: 11.4876x; 1.0008x over previous
import jax
import jax.numpy as jnp
from jax import lax
from jax.experimental import pallas as pl
from jax.experimental.pallas import tpu as pltpu, tpu_sc as plsc

N = 10000
E = 320000
D_IN = 128
D_HID = 128
D_OUT = 64
BN_EPS = 1e-3
ALPHA = 0.2

NC = 2
NS = 16
NW = NC * NS
NB = 5
K = 80
CH = 104
NFULL = N // CH
TAIL = N - NFULL * CH
DEGW = 16


def _make_sc_agg(d, split, with_deg):
    ept = E // NS if split else E // NW
    ncht = ept // K
    assert ncht % NB == 0
    mesh = plsc.VectorSubcoreMesh(core_axis_name="c", subcore_axis_name="s")
    out_type = [jax.ShapeDtypeStruct((NC, N, d), jnp.float32)]
    scratch = (
        [pltpu.VMEM((ncht, K), jnp.int32),
         pltpu.VMEM((ncht, K), jnp.int32),
         pltpu.VMEM((CH, d), jnp.float32),
         pltpu.VMEM_SHARED((N, d), jnp.float32)]
        + [pltpu.VMEM((K, d), jnp.float32)] * NB
        + [pltpu.SemaphoreType.DMA] * (2 * NB)
    )
    if with_deg:
        out_type.append(jax.ShapeDtypeStruct((NC, N, DEGW), jnp.float32))
        scratch += (
            [pltpu.VMEM((K, DEGW), jnp.float32),
             pltpu.VMEM((CH, DEGW), jnp.float32),
             pltpu.VMEM_SHARED((N, DEGW), jnp.float32)]
            + [pltpu.SemaphoreType.DMA] * NB
        )

    def body(table, src_h, dst_h, *rest):
        n_out = 2 if with_deg else 1
        if with_deg:
            out_agg, out_deg = rest[:2]
        else:
            (out_agg,) = rest[:1]
        rest = rest[n_out:]
        src_a, dst_a, zbuf, agg_sh = rest[:4]
        bufs = rest[4:4 + NB]
        gsem = rest[4 + NB:4 + 2 * NB]
        ssem = rest[4 + 2 * NB:4 + 3 * NB]
        if with_deg:
            ones_v, dzbuf, deg_sh = rest[4 + 3 * NB:7 + 3 * NB]
            dsem = rest[7 + 3 * NB:7 + 4 * NB]
        cid = lax.axis_index("c")
        sid = lax.axis_index("s")
        z16 = jnp.zeros((16,), jnp.float32)
        tbl = table.at[cid] if split else table
        crow = sid * ncht if split else (sid * NC + cid) * ncht

        pltpu.sync_copy(src_h.at[pl.ds(crow, ncht)], src_a)
        pltpu.sync_copy(dst_h.at[pl.ds(crow, ncht)], dst_a)

        def zrow(r, _):
            for c in range(d // 16):
                zbuf[r, pl.ds(c * 16, 16)] = z16
            return 0
        lax.fori_loop(0, CH, zrow, 0)
        for t in range(NFULL // NS):
            r0 = (sid + NS * t) * CH
            pltpu.sync_copy(zbuf, agg_sh.at[pl.ds(r0, CH)])

        @pl.when(sid == 0)
        def _zero_tail():
            pltpu.sync_copy(zbuf.at[pl.ds(0, TAIL)],
                            agg_sh.at[pl.ds(NFULL * CH, TAIL)])
        if with_deg:
            one16 = jnp.ones((16,), jnp.float32)

            def orow(r, _):
                ones_v[r] = one16
                return 0
            lax.fori_loop(0, K, orow, 0)

            def dzrow(r, _):
                dzbuf[r] = z16
                return 0
            lax.fori_loop(0, CH, dzrow, 0)
            for t in range(NFULL // NS):
                r0 = (sid + NS * t) * CH
                pltpu.sync_copy(dzbuf, deg_sh.at[pl.ds(r0, CH)])

            @pl.when(sid == 0)
            def _zero_deg_tail():
                pltpu.sync_copy(dzbuf.at[pl.ds(0, TAIL)],
                                deg_sh.at[pl.ds(NFULL * CH, TAIL)])
        plsc.subcore_barrier()

        for b in range(NB):
            pltpu.async_copy(tbl.at[src_a.at[b]], bufs[b], gsem[b])

        def blk(i, _):
            base = i * NB
            for b in range(NB):
                j = base + b
                pltpu.make_async_copy(tbl.at[src_a.at[j]], bufs[b],
                                      gsem[b]).wait()
                pltpu.async_copy(bufs[b], agg_sh.at[dst_a.at[j]], ssem[b],
                                 add=True)
                if with_deg:
                    pltpu.async_copy(ones_v, deg_sh.at[dst_a.at[j]],
                                     dsem[b], add=True)
            for b in range(NB):
                j = base + b
                pltpu.make_async_copy(bufs[b], agg_sh.at[dst_a.at[j]],
                                      ssem[b]).wait()
                if with_deg:
                    pltpu.make_async_copy(ones_v, deg_sh.at[dst_a.at[j]],
                                          dsem[b]).wait()
                jn = j + NB

                @pl.when(jn < ncht)
                def _next_gather():
                    pltpu.async_copy(tbl.at[src_a.at[jn]], bufs[b], gsem[b])
            return 0
        lax.fori_loop(0, ncht // NB, blk, 0)
        plsc.subcore_barrier()

        for t in range(NFULL // NS):
            r0 = (sid + NS * t) * CH
            pltpu.sync_copy(agg_sh.at[pl.ds(r0, CH)],
                            out_agg.at[cid, pl.ds(r0, CH)])

        @pl.when(sid == 0)
        def _write_tail():
            pltpu.sync_copy(agg_sh.at[pl.ds(NFULL * CH, TAIL)],
                            out_agg.at[cid, pl.ds(NFULL * CH, TAIL)])
        if with_deg:
            for t in range(NFULL // NS):
                r0 = (sid + NS * t) * CH
                pltpu.sync_copy(deg_sh.at[pl.ds(r0, CH)],
                                out_deg.at[cid, pl.ds(r0, CH)])

            @pl.when(sid == 0)
            def _write_deg_tail():
                pltpu.sync_copy(deg_sh.at[pl.ds(NFULL * CH, TAIL)],
                                out_deg.at[cid, pl.ds(NFULL * CH, TAIL)])

    return pl.kernel(
        body, out_type=out_type, mesh=mesh, scratch_types=scratch,
        compiler_params=pltpu.CompilerParams(use_tc_tiling_on_sc=False))


_sc_agg_l1 = _make_sc_agg(D_IN // 2, True, True)
_sc_agg_l2 = _make_sc_agg(D_OUT, False, False)

_HI = lax.Precision.HIGHEST


def _dot(a, b):
    return jnp.dot(a, b, precision=_HI, preferred_element_type=jnp.float32)


def _dense1(x, aggp, degp, A1, B1a, B1b, c1, B2):
    R = 1000
    G = N // R

    def body(x_ref, agg_ref, deg_ref, a_ref, b1a_ref, b1b_ref, c_ref,
             w2_ref, h_ref, hw_ref):
        deg = jnp.maximum(deg_ref[0][:, 0:1], 1.0)
        m0 = agg_ref[0] / deg
        m1 = agg_ref[1] / deg
        h = (_dot(x_ref[...], a_ref[...]) + _dot(m0, b1a_ref[...])
             + _dot(m1, b1b_ref[...]) + c_ref[...])
        h = jnp.where(h >= 0, h, ALPHA * h)
        h_ref[...] = h
        hw_ref[...] = _dot(h, w2_ref[...])

    return pl.pallas_call(
        body,
        grid=(G,),
        in_specs=[
            pl.BlockSpec((R, D_IN), lambda i: (i, 0)),
            pl.BlockSpec((NC, R, D_IN // 2), lambda i: (0, i, 0)),
            pl.BlockSpec((1, R, DEGW), lambda i: (0, i, 0)),
            pl.BlockSpec((D_IN, D_HID), lambda i: (0, 0)),
            pl.BlockSpec((D_IN // 2, D_HID), lambda i: (0, 0)),
            pl.BlockSpec((D_IN // 2, D_HID), lambda i: (0, 0)),
            pl.BlockSpec((1, D_HID), lambda i: (0, 0)),
            pl.BlockSpec((D_HID, D_OUT), lambda i: (0, 0)),
        ],
        out_specs=[
            pl.BlockSpec((R, D_HID), lambda i: (i, 0)),
            pl.BlockSpec((R, D_OUT), lambda i: (i, 0)),
        ],
        out_shape=[
            jax.ShapeDtypeStruct((N, D_HID), jnp.float32),
            jax.ShapeDtypeStruct((N, D_OUT), jnp.float32),
        ],
    )(x, aggp, degp, A1, B1a, B1b, c1, B2)


def _dense2(h, aggp2, degp, A2, c2):
    R = 1000
    G = N // R

    def body(h_ref, agg_ref, deg_ref, a_ref, c_ref, o_ref):
        deg = jnp.maximum(deg_ref[0][:, 0:1], 1.0)
        mean = (agg_ref[0] + agg_ref[1]) / deg
        y = _dot(h_ref[...], a_ref[...]) + mean + c_ref[...]
        o_ref[...] = jnp.where(y >= 0, y, ALPHA * y)

    return pl.pallas_call(
        body,
        grid=(G,),
        in_specs=[
            pl.BlockSpec((R, D_HID), lambda i: (i, 0)),
            pl.BlockSpec((NC, R, D_OUT), lambda i: (0, i, 0)),
            pl.BlockSpec((1, R, DEGW), lambda i: (0, i, 0)),
            pl.BlockSpec((D_HID, D_OUT), lambda i: (0, 0)),
            pl.BlockSpec((1, D_OUT), lambda i: (0, 0)),
        ],
        out_specs=pl.BlockSpec((R, D_OUT), lambda i: (i, 0)),
        out_shape=jax.ShapeDtypeStruct((N, D_OUT), jnp.float32),
    )(h, aggp2, degp, A2, c2)


def kernel(x, edge_index, W_self1, W_neigh1, b1, gamma1, beta1,
           W_self2, W_neigh2, b2, gamma2, beta2):
    s1 = gamma1 * lax.rsqrt(jnp.float32(1.0 + BN_EPS))
    A1 = W_self1 * s1[None, :]
    B1 = W_neigh1 * s1[None, :]
    c1 = (b1 * s1 + beta1)[None, :]
    s2 = gamma2 * lax.rsqrt(jnp.float32(1.0 + BN_EPS))
    A2 = W_self2 * s2[None, :]
    B2 = W_neigh2 * s2[None, :]
    c2 = (b2 * s2 + beta2)[None, :]

    src = edge_index[0].astype(jnp.int32).reshape(E // K, K)
    dst = edge_index[1].astype(jnp.int32).reshape(E // K, K)

    xs = x.reshape(N, NC, D_IN // 2).transpose(1, 0, 2)

    aggp1, degp = _sc_agg_l1(xs, src, dst)
    h, hw = _dense1(x, aggp1, degp, A1, B1[: D_IN // 2], B1[D_IN // 2:],
                    c1, B2)
    [aggp2] = _sc_agg_l2(hw, src, dst)
    out = _dense2(h, aggp2, degp, A2, c2)
    return out
